# baseline (speedup 1.0000x reference)
import functools

import jax
import jax.numpy as jnp
from jax import lax
from jax.experimental import pallas as pl
from jax.experimental.pallas import tpu as pltpu
from jax.experimental.pallas import tpu_sc as plsc

NC = 2
NS = 16
DD = 16


def _sc_aggregate(x, edge_index, zf, zd, *, n, e, d):
    nw = NC * NS
    ept = e // nw
    ch = 40
    nch = ept // ch
    nbuf = 5
    rpt = n // NS
    rb = zf.shape[0]
    nrb = rpt // rb

    mesh = plsc.VectorSubcoreMesh(
        core_axis_name="c", subcore_axis_name="s", num_cores=NC, num_subcores=NS
    )

    @functools.partial(
        pl.kernel,
        out_type=(
            jax.ShapeDtypeStruct((NC, n, d), jnp.float32),
            jax.ShapeDtypeStruct((NC, n, DD), jnp.float32),
        ),
        mesh=mesh,
        compiler_params=pltpu.CompilerParams(use_tc_tiling_on_sc=False),
        scratch_types=[
            pltpu.VMEM_SHARED((n, d), jnp.float32),
            pltpu.VMEM_SHARED((n, DD), jnp.float32),
            pltpu.VMEM((ept,), jnp.int32),
            pltpu.VMEM((ch, DD), jnp.float32),
            pltpu.VMEM((32, 128), jnp.float32),
            [pltpu.VMEM((ch, d), jnp.float32) for _ in range(nbuf)],
            [pltpu.VMEM((ch,), jnp.int32) for _ in range(nbuf)],
            [pltpu.SemaphoreType.DMA for _ in range(nbuf)],
            [pltpu.SemaphoreType.DMA for _ in range(nbuf)],
            [pltpu.SemaphoreType.DMA for _ in range(nbuf)],
            [pltpu.SemaphoreType.DMA for _ in range(nbuf)],
        ],
    )
    def k(x_hbm, ei_hbm, zf_hbm, zd_hbm, outf_hbm, outd_hbm,
          accf, accd, colbuf, ones, zbuf, rows, ridx, gsem, isem, ssem, osem):
        c = lax.axis_index("c")
        s = lax.axis_index("s")
        wid = c * NS + s
        ebase = wid * ept
        pltpu.sync_copy(ei_hbm.at[1, pl.ds(ebase, ept)], colbuf)

        def fetch(kk, b):
            pltpu.async_copy(ei_hbm.at[0, pl.ds(ebase + kk * ch, ch)], ridx[b], isem[b])
            pltpu.async_copy(x_hbm.at[colbuf.at[pl.ds(kk * ch, ch)]], rows[b], gsem[b])

        def drain_fetch(b):
            pltpu.make_async_copy(ei_hbm.at[0, pl.ds(0, ch)], ridx[b], isem[b]).wait()
            pltpu.make_async_copy(x_hbm.at[pl.ds(0, ch)], rows[b], gsem[b]).wait()

        for b in range(nbuf):
            fetch(b, b)
        one_row = jnp.where(
            lax.broadcasted_iota(jnp.int32, (16,), 0) == 0, 1.0, 0.0
        ).astype(jnp.float32)
        zero_row = jnp.zeros((16,), jnp.float32)
        for i in range(ch):
            ones[i, pl.ds(0, 16)] = one_row
        for i in range(32):
            for j in range(8):
                zbuf[i, pl.ds(j * 16, 16)] = zero_row
        zb = 32
        nzb = rpt // zb
        for r in range(nzb):
            pltpu.sync_copy(zbuf, accf.at[pl.ds(s * rpt + r * zb, zb)])
        if rpt % zb:
            pltpu.sync_copy(
                zbuf.at[pl.ds(0, rpt % zb)],
                accf.at[pl.ds(s * rpt + nzb * zb, rpt % zb)],
            )
        for r in range(nrb):
            pltpu.sync_copy(zd_hbm, accd.at[pl.ds(s * rpt + r * rb, rb)])
        plsc.subcore_barrier()

        def body(i, carry):
            descs = []
            for b in range(nbuf):
                drain_fetch(b)
                descs.append((
                    pltpu.async_copy(rows[b], accf.at[ridx[b]], ssem[b], add=True),
                    pltpu.async_copy(ones, accd.at[ridx[b]], osem[b], add=True),
                ))
            for b in range(nbuf):
                kk = (i + 1) * nbuf + b
                descs[b][0].wait()
                descs[b][1].wait()

                @pl.when(kk < nch)
                def _():
                    fetch(kk, b)

            return carry

        lax.fori_loop(0, nch // nbuf, body, 0)
        for kk in range(nch - nch % nbuf, nch):
            b = kk % nbuf
            drain_fetch(b)
            pltpu.sync_copy(rows[b], accf.at[ridx[b]], add=True)
            pltpu.sync_copy(ones, accd.at[ridx[b]], add=True)
        plsc.subcore_barrier()
        for r in range(nrb):
            base = s * rpt + r * rb
            pltpu.sync_copy(accf.at[pl.ds(base, rb)], outf_hbm.at[c, pl.ds(base, rb)])
            pltpu.sync_copy(accd.at[pl.ds(base, rb)], outd_hbm.at[c, pl.ds(base, rb)])

    return k(x, edge_index, zf, zd)


def _tc_finish(pf, degb, w, bias2, *, n, d_out):
    blk = 2000

    def body(p0_ref, p1_ref, d_ref, w_ref, b_ref, o_ref):
        feat = (p0_ref[...] + p1_ref[...]).reshape(blk, 128)
        deg = d_ref[...]
        dinv = jnp.where(deg > 0, lax.rsqrt(deg), 0.0)
        o_ref[...] = (
            jnp.dot(feat * dinv, w_ref[...], preferred_element_type=jnp.float32)
            + b_ref[...]
        )

    return pl.pallas_call(
        body,
        grid=(n // blk,),
        in_specs=[
            pl.BlockSpec((blk * 128,), lambda i: (i,)),
            pl.BlockSpec((blk * 128,), lambda i: (i + n // blk,)),
            pl.BlockSpec((blk, 128), lambda i: (i, 0)),
            pl.BlockSpec(w.shape, lambda i: (0, 0)),
            pl.BlockSpec((1, d_out), lambda i: (0, 0)),
        ],
        out_specs=pl.BlockSpec((blk, d_out), lambda i: (i, 0)),
        out_shape=jax.ShapeDtypeStruct((n, d_out), jnp.float32),
    )(pf.reshape(-1), pf.reshape(-1), degb, w, bias2)


def kernel(x, edge_index, size, w, bias):
    n, d = x.shape
    e = edge_index.shape[1]
    d_out = w.shape[1]
    zf = jnp.zeros((125, d), jnp.float32)
    zd = jnp.zeros((125, DD), jnp.float32)
    pf, pd = _sc_aggregate(x, edge_index, zf, zd, n=n, e=e, d=d)
    degb = jnp.broadcast_to((pd[0, :, 0] + pd[1, :, 0])[:, None], (n, 128))
    shift = (jnp.asarray(size) - n).astype(x.dtype)
    bias2 = (bias + shift).reshape(1, d_out)
    return _tc_finish(pf, degb, w, bias2, n=n, d_out=d_out)


# speedup vs baseline: 1.0044x; 1.0044x over previous
import functools

import jax
import jax.numpy as jnp
from jax import lax
from jax.experimental import pallas as pl
from jax.experimental.pallas import tpu as pltpu
from jax.experimental.pallas import tpu_sc as plsc

NC = 2
NS = 16
DD = 16


def _sc_aggregate(x, edge_index, zd, *, n, e, d):
    nw = NC * NS
    ept = e // nw
    ch = 40
    nch = ept // ch
    nbuf = 5
    rpt = n // NS
    rb = zd.shape[0]
    nrb = rpt // rb

    mesh = plsc.VectorSubcoreMesh(
        core_axis_name="c", subcore_axis_name="s", num_cores=NC, num_subcores=NS
    )

    @functools.partial(
        pl.kernel,
        out_type=(
            jax.ShapeDtypeStruct((NC, n, d), jnp.float32),
            jax.ShapeDtypeStruct((NC, n, DD), jnp.float32),
        ),
        mesh=mesh,
        compiler_params=pltpu.CompilerParams(use_tc_tiling_on_sc=False),
        scratch_types=[
            pltpu.VMEM_SHARED((n, d), jnp.float32),
            pltpu.VMEM_SHARED((n, DD), jnp.float32),
            pltpu.VMEM((ept,), jnp.int32),
            pltpu.VMEM((ch, DD), jnp.float32),
            pltpu.VMEM((32, 128), jnp.float32),
            [pltpu.VMEM((ch, d), jnp.float32) for _ in range(nbuf)],
            [pltpu.VMEM((ch,), jnp.int32) for _ in range(nbuf)],
            [pltpu.SemaphoreType.DMA for _ in range(nbuf)],
            [pltpu.SemaphoreType.DMA for _ in range(nbuf)],
            [pltpu.SemaphoreType.DMA for _ in range(nbuf)],
            [pltpu.SemaphoreType.DMA for _ in range(nbuf)],
        ],
    )
    def k(x_hbm, ei_hbm, zd_hbm, outf_hbm, outd_hbm,
          accf, accd, colbuf, ones, zbuf, rows, ridx, gsem, isem, ssem, osem):
        c = lax.axis_index("c")
        s = lax.axis_index("s")
        wid = c * NS + s
        ebase = wid * ept
        pltpu.sync_copy(ei_hbm.at[1, pl.ds(ebase, ept)], colbuf)

        def fetch(kk, b):
            pltpu.async_copy(ei_hbm.at[0, pl.ds(ebase + kk * ch, ch)], ridx[b], isem[b])
            pltpu.async_copy(x_hbm.at[colbuf.at[pl.ds(kk * ch, ch)]], rows[b], gsem[b])

        def drain_fetch(b):
            pltpu.make_async_copy(ei_hbm.at[0, pl.ds(0, ch)], ridx[b], isem[b]).wait()
            pltpu.make_async_copy(x_hbm.at[pl.ds(0, ch)], rows[b], gsem[b]).wait()

        for b in range(nbuf):
            fetch(b, b)
        one_row = jnp.where(
            lax.broadcasted_iota(jnp.int32, (16,), 0) == 0, 1.0, 0.0
        ).astype(jnp.float32)
        zero_row = jnp.zeros((16,), jnp.float32)
        for i in range(ch):
            ones[i, pl.ds(0, 16)] = one_row
        for i in range(32):
            for j in range(8):
                zbuf[i, pl.ds(j * 16, 16)] = zero_row
        zb = 32
        nzb = rpt // zb
        for r in range(nzb):
            pltpu.sync_copy(zbuf, accf.at[pl.ds(s * rpt + r * zb, zb)])
        if rpt % zb:
            pltpu.sync_copy(
                zbuf.at[pl.ds(0, rpt % zb)],
                accf.at[pl.ds(s * rpt + nzb * zb, rpt % zb)],
            )
        for r in range(nrb):
            pltpu.sync_copy(zd_hbm, accd.at[pl.ds(s * rpt + r * rb, rb)])
        plsc.subcore_barrier()

        def body(i, carry):
            descs = []
            for b in range(nbuf):
                drain_fetch(b)
                descs.append((
                    pltpu.async_copy(rows[b], accf.at[ridx[b]], ssem[b], add=True),
                    pltpu.async_copy(ones, accd.at[ridx[b]], osem[b], add=True),
                ))
            for b in range(nbuf):
                kk = (i + 1) * nbuf + b
                descs[b][0].wait()
                descs[b][1].wait()

                @pl.when(kk < nch)
                def _():
                    fetch(kk, b)

            return carry

        lax.fori_loop(0, nch // nbuf, body, 0)
        for kk in range(nch - nch % nbuf, nch):
            b = kk % nbuf
            drain_fetch(b)
            pltpu.sync_copy(rows[b], accf.at[ridx[b]], add=True)
            pltpu.sync_copy(ones, accd.at[ridx[b]], add=True)
        plsc.subcore_barrier()
        for r in range(nrb):
            base = s * rpt + r * rb
            pltpu.sync_copy(accf.at[pl.ds(base, rb)], outf_hbm.at[c, pl.ds(base, rb)])
            pltpu.sync_copy(accd.at[pl.ds(base, rb)], outd_hbm.at[c, pl.ds(base, rb)])

    return k(x, edge_index, zd)


def _tc_finish(pf, degb, w, bias2, *, n, d_out):
    blk = 2000

    def body(p0_ref, p1_ref, d_ref, w_ref, b_ref, o_ref):
        feat = (p0_ref[...] + p1_ref[...]).reshape(blk, 128)
        deg = d_ref[...]
        dinv = jnp.where(deg > 0, lax.rsqrt(deg), 0.0)
        o_ref[...] = (
            jnp.dot(feat * dinv, w_ref[...], preferred_element_type=jnp.float32)
            + b_ref[...]
        )

    return pl.pallas_call(
        body,
        grid=(n // blk,),
        in_specs=[
            pl.BlockSpec((blk * 128,), lambda i: (i,)),
            pl.BlockSpec((blk * 128,), lambda i: (i + n // blk,)),
            pl.BlockSpec((blk, 128), lambda i: (i, 0)),
            pl.BlockSpec(w.shape, lambda i: (0, 0)),
            pl.BlockSpec((1, d_out), lambda i: (0, 0)),
        ],
        out_specs=pl.BlockSpec((blk, d_out), lambda i: (i, 0)),
        out_shape=jax.ShapeDtypeStruct((n, d_out), jnp.float32),
    )(pf.reshape(-1), pf.reshape(-1), degb, w, bias2)


def kernel(x, edge_index, size, w, bias):
    n, d = x.shape
    e = edge_index.shape[1]
    d_out = w.shape[1]
    zd = jnp.zeros((125, DD), jnp.float32)
    pf, pd = _sc_aggregate(x, edge_index, zd, n=n, e=e, d=d)
    degb = jnp.broadcast_to((pd[0, :, 0] + pd[1, :, 0])[:, None], (n, 128))
    shift = (jnp.asarray(size) - n).astype(x.dtype)
    bias2 = (bias + shift).reshape(1, d_out)
    return _tc_finish(pf, degb, w, bias2, n=n, d_out=d_out)
